# baseline (speedup 1.0000x reference)
import functools

import jax
import jax.numpy as jnp
from jax import lax
from jax.experimental import pallas as pl
from jax.experimental.pallas import tpu as pltpu
from jax.experimental.pallas import tpu_sc as plsc

EMB_IN = 50
EMB_OUT = 128
NW = 32
CHUNK = 128
NSLOT = 3
LANES = 16


def _mm_body(emb_ref, w_ref, b_ref, out_ref):
    out_ref[...] = (
        jnp.dot(emb_ref[:, :EMB_IN], w_ref[...],
                preferred_element_type=jnp.float32)
        + b_ref[...]
    )


def _tc_project(emb, W, b):
    n = emb.shape[0]
    bm = 2048
    mm = pl.pallas_call(
        _mm_body,
        grid=(n // bm,),
        in_specs=[
            pl.BlockSpec((bm, EMB_OUT), lambda i: (i, 0)),
            pl.BlockSpec((EMB_IN, EMB_OUT), lambda i: (0, 0)),
            pl.BlockSpec((1, EMB_OUT), lambda i: (0, 0)),
        ],
        out_specs=pl.BlockSpec((bm, EMB_OUT), lambda i: (i, 0)),
        out_shape=jax.ShapeDtypeStruct((n, EMB_OUT), jnp.float32),
    )
    return mm(emb, W, b.reshape(1, EMB_OUT))


def _gx_body(table_hbm, jp_hbm, e_hbm, out_hbm,
             jp_v, e_v, gbuf, obuf, gsem, osem, *, nchunk):
    wid = lax.axis_index("s") * 2 + lax.axis_index("c")
    per_w = nchunk * CHUNK

    pltpu.sync_copy(jp_hbm.at[pl.ds(wid * 2 * nchunk, 2 * nchunk)], jp_v)
    pltpu.sync_copy(e_hbm.at[pl.ds(wid * nchunk, nchunk)], e_v)

    def gather_start(c, slot):
        pltpu.async_copy(table_hbm.at[jp_v.at[2 * c]],
                         gbuf.at[slot, 0], gsem.at[slot])
        pltpu.async_copy(table_hbm.at[jp_v.at[2 * c + 1]],
                         gbuf.at[slot, 1], gsem.at[slot])

    def gather_wait(slot):
        pltpu.make_async_copy(table_hbm.at[jp_v.at[0]],
                              gbuf.at[slot, 0], gsem.at[slot]).wait()
        pltpu.make_async_copy(table_hbm.at[jp_v.at[0]],
                              gbuf.at[slot, 1], gsem.at[slot]).wait()

    def out_start(c, slot):
        base = wid * per_w + c * CHUNK
        pltpu.async_copy(obuf.at[slot], out_hbm.at[pl.ds(base, CHUNK)],
                         osem.at[slot])

    def out_wait(slot):
        pltpu.make_async_copy(obuf.at[slot],
                              out_hbm.at[pl.ds(0, CHUNK)],
                              osem.at[slot]).wait()

    iota = lax.iota(jnp.int32, LANES)

    def extract(c, slot):
        src = gbuf.at[slot]
        dst = obuf.at[slot]

        def group(g, _):
            r0 = g * LANES
            rows = iota + r0
            evec = e_v[c, pl.ds(r0, LANES)]
            tbase = rows * EMB_IN + evec
            for k in range(EMB_IN):
                pos = evec + k
                d0 = jnp.where(pos >= EMB_IN, 1, 0)
                t = tbase + k - d0 * EMB_IN
                d1 = (t * 18725) >> 20
                d2 = t - d1 * 56
                val = plsc.load_gather(src, [d0, d1, d2])
                plsc.store_scatter(
                    dst, [rows, jnp.full((LANES,), k, jnp.int32)], val)
            return 0

        lax.fori_loop(0, CHUNK // LANES, group, 0)

    gather_start(0, 0)
    gather_start(1, 1)

    def body(c, _):
        slot = lax.rem(c, NSLOT)
        gather_wait(slot)

        @pl.when(c >= NSLOT)
        def _():
            out_wait(slot)

        extract(c, slot)
        out_start(c, slot)

        @pl.when(c + 2 < nchunk)
        def _():
            gather_start(c + 2, lax.rem(c + 2, NSLOT))
        return 0

    lax.fori_loop(0, nchunk, body, 0)

    for c in range(nchunk - NSLOT, nchunk):
        out_wait(c % NSLOT)


def _sc_gather_extract(table, jp, e2):
    nrow = e2.shape[0] * CHUNK
    nchunk = e2.shape[0] // NW
    mesh = plsc.VectorSubcoreMesh(core_axis_name="c", subcore_axis_name="s")
    run = pl.kernel(
        functools.partial(_gx_body, nchunk=nchunk),
        out_type=jax.ShapeDtypeStruct((nrow, EMB_OUT), jnp.float32),
        scratch_types=[
            pltpu.VMEM((2 * nchunk, CHUNK), jnp.int32),
            pltpu.VMEM((nchunk, CHUNK), jnp.int32),
            pltpu.VMEM((NSLOT, 2, CHUNK, EMB_IN), jnp.float32),
            pltpu.VMEM((NSLOT, CHUNK, EMB_OUT), jnp.float32),
            pltpu.SemaphoreType.DMA((NSLOT,)),
            pltpu.SemaphoreType.DMA((NSLOT,)),
        ],
        mesh=mesh,
        compiler_params=pltpu.CompilerParams(use_tc_tiling_on_sc=False, needs_layout_passes=False),
    )
    return run(table, jp, e2)


def kernel(observations, table, W, b):
    batch, seq = observations.shape
    n = batch * seq
    i = observations.reshape(-1).astype(jnp.int32)
    j1 = (i * 56) // EMB_IN
    e = i * 56 - j1 * EMB_IN
    j1_2 = j1.reshape(n // CHUNK, CHUNK)
    jp = jnp.stack([j1_2, j1_2 + 1], axis=1).reshape(2 * n // CHUNK, CHUNK)
    e2 = e.reshape(n // CHUNK, CHUNK)
    emb = _sc_gather_extract(table, jp, e2)
    out = _tc_project(emb, W, b)
    return out.reshape(batch, seq, EMB_OUT)


# speedup vs baseline: 2.6804x; 2.6804x over previous
import functools

import jax
import jax.numpy as jnp
from jax import lax
from jax.experimental import pallas as pl
from jax.experimental.pallas import tpu as pltpu
from jax.experimental.pallas import tpu_sc as plsc

EMB_IN = 50
EMB_OUT = 128
NW = 32
CHUNK = 128


def _proj_body(t_ref, w_ref, b_ref, p_ref):
    p_ref[...] = (
        jnp.dot(t_ref[...], w_ref[...], preferred_element_type=jnp.float32)
        + b_ref[...]
    )


def _tc_project_table(table, W, b):
    v = table.shape[0]
    bm = 8000
    mm = pl.pallas_call(
        _proj_body,
        grid=(v // bm,),
        in_specs=[
            pl.BlockSpec((bm, EMB_IN), lambda i: (i, 0)),
            pl.BlockSpec((EMB_IN, EMB_OUT), lambda i: (0, 0)),
            pl.BlockSpec((1, EMB_OUT), lambda i: (0, 0)),
        ],
        out_specs=pl.BlockSpec((bm, EMB_OUT), lambda i: (i, 0)),
        out_shape=jax.ShapeDtypeStruct((v, EMB_OUT), jnp.float32),
    )
    return mm(table, W, b.reshape(1, EMB_OUT))


NSLOT = 6


def _gather_body(p_hbm, idx_hbm, out_hbm, idx_v, buf_v, gsem, osem, *, nchunk):
    wid = lax.axis_index("s") * 2 + lax.axis_index("c")
    per_w = nchunk * CHUNK
    pltpu.sync_copy(idx_hbm.at[pl.ds(wid * nchunk, nchunk)], idx_v)

    def gather_start(j, slot):
        pltpu.async_copy(p_hbm.at[idx_v.at[j]], buf_v.at[slot], gsem.at[slot])

    def gather_wait(slot):
        pltpu.make_async_copy(p_hbm.at[idx_v.at[0]], buf_v.at[slot],
                              gsem.at[slot]).wait()

    def out_start(j, slot):
        base = wid * per_w + j * CHUNK
        pltpu.async_copy(buf_v.at[slot], out_hbm.at[pl.ds(base, CHUNK)],
                         osem.at[slot])

    def out_wait(slot):
        base = wid * per_w
        pltpu.make_async_copy(buf_v.at[slot], out_hbm.at[pl.ds(base, CHUNK)],
                              osem.at[slot]).wait()

    for j in range(NSLOT - 1):
        gather_start(j, j)

    def body(j, _):
        slot = lax.rem(j, NSLOT)
        gather_wait(slot)
        out_start(j, slot)

        @pl.when(j + NSLOT - 1 < nchunk)
        def _():
            s2 = lax.rem(j + NSLOT - 1, NSLOT)

            @pl.when(j > 0)
            def _():
                out_wait(s2)
            gather_start(j + NSLOT - 1, s2)

        return 0

    lax.fori_loop(0, nchunk, body, 0)

    for k in range(nchunk - NSLOT, nchunk):
        out_wait(k % NSLOT)


def _sc_gather(p, idx2):
    nrow = idx2.shape[0] * CHUNK
    nchunk = idx2.shape[0] // NW
    mesh = plsc.VectorSubcoreMesh(core_axis_name="c", subcore_axis_name="s")
    gather = pl.kernel(
        functools.partial(_gather_body, nchunk=nchunk),
        out_type=jax.ShapeDtypeStruct((nrow, EMB_OUT), jnp.float32),
        scratch_types=[
            pltpu.VMEM((nchunk, CHUNK), jnp.int32),
            pltpu.VMEM((NSLOT, CHUNK, EMB_OUT), jnp.float32),
            pltpu.SemaphoreType.DMA((NSLOT,)),
            pltpu.SemaphoreType.DMA((NSLOT,)),
        ],
        mesh=mesh,
        compiler_params=pltpu.CompilerParams(use_tc_tiling_on_sc=False),
    )
    return gather(p, idx2)


def kernel(observations, table, W, b):
    batch, seq = observations.shape
    n = batch * seq
    p = _tc_project_table(table, W, b)
    idx2 = observations.reshape(n // CHUNK, CHUNK).astype(jnp.int32)
    out = _sc_gather(p, idx2)
    return out.reshape(batch, seq, EMB_OUT)


# speedup vs baseline: 2.6905x; 1.0038x over previous
import functools

import jax
import jax.numpy as jnp
from jax import lax
from jax.experimental import pallas as pl
from jax.experimental.pallas import tpu as pltpu
from jax.experimental.pallas import tpu_sc as plsc

EMB_IN = 50
EMB_OUT = 128
NW = 32
CHUNK = 128


def _proj_body(t_ref, w_ref, b_ref, p_ref):
    p_ref[...] = (
        jnp.dot(t_ref[...], w_ref[...], preferred_element_type=jnp.float32)
        + b_ref[...]
    )


def _tc_project_table(table, W, b):
    v = table.shape[0]
    bm = 20000
    mm = pl.pallas_call(
        _proj_body,
        grid=(v // bm,),
        in_specs=[
            pl.BlockSpec((bm, EMB_IN), lambda i: (i, 0)),
            pl.BlockSpec((EMB_IN, EMB_OUT), lambda i: (0, 0)),
            pl.BlockSpec((1, EMB_OUT), lambda i: (0, 0)),
        ],
        out_specs=pl.BlockSpec((bm, EMB_OUT), lambda i: (i, 0)),
        out_shape=jax.ShapeDtypeStruct((v, EMB_OUT), jnp.float32),
    )
    return mm(table, W, b.reshape(1, EMB_OUT))


NSLOT = 6


def _gather_body(p_hbm, idx_hbm, out_hbm, idx_v, buf_v, gsem, osem, *, nchunk):
    wid = lax.axis_index("s") * 2 + lax.axis_index("c")
    per_w = nchunk * CHUNK
    pltpu.sync_copy(idx_hbm.at[pl.ds(wid * nchunk, nchunk)], idx_v)

    def gather_start(j, slot):
        pltpu.async_copy(p_hbm.at[idx_v.at[j]], buf_v.at[slot], gsem.at[slot])

    def gather_wait(slot):
        pltpu.make_async_copy(p_hbm.at[idx_v.at[0]], buf_v.at[slot],
                              gsem.at[slot]).wait()

    def out_start(j, slot):
        base = wid * per_w + j * CHUNK
        pltpu.async_copy(buf_v.at[slot], out_hbm.at[pl.ds(base, CHUNK)],
                         osem.at[slot])

    def out_wait(slot):
        base = wid * per_w
        pltpu.make_async_copy(buf_v.at[slot], out_hbm.at[pl.ds(base, CHUNK)],
                              osem.at[slot]).wait()

    for j in range(NSLOT - 1):
        gather_start(j, j)

    def body(j, _):
        slot = lax.rem(j, NSLOT)
        gather_wait(slot)
        out_start(j, slot)

        @pl.when(j + NSLOT - 1 < nchunk)
        def _():
            s2 = lax.rem(j + NSLOT - 1, NSLOT)

            @pl.when(j > 0)
            def _():
                out_wait(s2)
            gather_start(j + NSLOT - 1, s2)

        return 0

    lax.fori_loop(0, nchunk, body, 0)

    for k in range(nchunk - NSLOT, nchunk):
        out_wait(k % NSLOT)


def _sc_gather(p, idx2):
    nrow = idx2.shape[0] * CHUNK
    nchunk = idx2.shape[0] // NW
    mesh = plsc.VectorSubcoreMesh(core_axis_name="c", subcore_axis_name="s")
    gather = pl.kernel(
        functools.partial(_gather_body, nchunk=nchunk),
        out_type=jax.ShapeDtypeStruct((nrow, EMB_OUT), jnp.float32),
        scratch_types=[
            pltpu.VMEM((nchunk, CHUNK), jnp.int32),
            pltpu.VMEM((NSLOT, CHUNK, EMB_OUT), jnp.float32),
            pltpu.SemaphoreType.DMA((NSLOT,)),
            pltpu.SemaphoreType.DMA((NSLOT,)),
        ],
        mesh=mesh,
        compiler_params=pltpu.CompilerParams(use_tc_tiling_on_sc=False),
    )
    return gather(p, idx2)


def kernel(observations, table, W, b):
    batch, seq = observations.shape
    n = batch * seq
    p = _tc_project_table(table, W, b)
    idx2 = observations.reshape(n // CHUNK, CHUNK).astype(jnp.int32)
    out = _sc_gather(p, idx2)
    return out.reshape(batch, seq, EMB_OUT)
